# baseline (device time: 62571 ns/iter reference)
import jax
import jax.numpy as jnp
from jax import lax
from jax.experimental import pallas as pl
from jax.experimental.pallas import tpu as pltpu

N_DEV = 8
HEADS_PER = 8
SQ = 256
HALF = SQ // 2
SKV = 4096
DH = 128
DM = HEADS_PER * DH
BLK = 64
SCALE = 0.08838834764831843


def kernel(x, Wq, K_ext, V_ext, Wo):
    XOR_STAGES = (1, 3, 4)

    def body(x_ref, wq_ref, k_any, v_any, wo_ref, out_ref,
             kv_bufs, kv_sems, send_ref, recv_ref, send_sems, recv_sems):
        my_pos = lax.axis_index("i")
        partners = [jnp.bitwise_xor(my_pos, c) for c in XOR_STAGES]

        barrier_sem = pltpu.get_barrier_semaphore()
        for nbr in partners:
            pl.semaphore_signal(
                barrier_sem, inc=1,
                device_id=(nbr,), device_id_type=pl.DeviceIdType.MESH,
            )
        pl.semaphore_wait(barrier_sem, len(partners))

        def start_kv(h, slot):
            head = my_pos * HEADS_PER + h
            kcp = pltpu.make_async_copy(
                k_any.at[0, :, head, :], kv_bufs.at[slot, 0],
                kv_sems.at[slot, 0])
            vcp = pltpu.make_async_copy(
                v_any.at[0, :, head, :], kv_bufs.at[slot, 1],
                kv_sems.at[slot, 1])
            kcp.start()
            vcp.start()
            return kcp, vcp

        pending = start_kv(0, 0)

        xb = x_ref[0].astype(jnp.bfloat16)
        wqb = wq_ref[...].astype(jnp.bfloat16)
        wob = wo_ref[...].astype(jnp.bfloat16)
        q = jnp.dot(xb, wqb, preferred_element_type=jnp.float32)
        qs = (q * SCALE).astype(jnp.bfloat16)

        def make_bias(row0):
            rowb = (lax.broadcasted_iota(jnp.int32, (HALF, SKV), 0)
                    + row0) // BLK
            colb = lax.broadcasted_iota(jnp.int32, (HALF, SKV), 1) // BLK
            keep = (rowb == colb) | (colb == 0) | (
                lax.rem(rowb + colb, 3) == 0)
            return jnp.where(keep, 0.0, -1e9).astype(jnp.float32)

        biases = [make_bias(0), make_bias(HALF)]

        def chunk_head_partial(c, h, kh, vh):
            qh = qs[c * HALF:(c + 1) * HALF, h * DH:(h + 1) * DH]
            s = lax.dot_general(
                qh, kh, (((1,), (1,)), ((), ())),
                preferred_element_type=jnp.float32)
            w = jnp.exp(s + biases[c])
            denom = jnp.sum(w, axis=-1, keepdims=True)
            ctx = jnp.dot(w.astype(jnp.bfloat16), vh,
                          preferred_element_type=jnp.float32)
            ctx = (ctx / denom).astype(jnp.bfloat16)
            return jnp.dot(ctx, wob[h * DH:(h + 1) * DH, :],
                           preferred_element_type=jnp.float32)

        def compute_chunk(c, i0):
            nonlocal pending
            acc = jnp.zeros((HALF, DM), jnp.float32)
            for h in range(HEADS_PER):
                i = i0 + h
                slot = i % 2
                pending[0].wait()
                pending[1].wait()
                if i + 1 < 2 * HEADS_PER:
                    pending = start_kv((h + 1) % HEADS_PER, (i + 1) % 2)
                kh = kv_bufs[slot, 0].astype(jnp.bfloat16)
                vh = kv_bufs[slot, 1].astype(jnp.bfloat16)
                acc = acc + chunk_head_partial(c, h, kh, vh)
            return acc

        def exchange(s, c, acc):
            send_ref[s, c] = acc.astype(jnp.bfloat16)
            rdma = pltpu.make_async_remote_copy(
                src_ref=send_ref.at[s, c],
                dst_ref=recv_ref.at[s, c],
                send_sem=send_sems.at[s, c],
                recv_sem=recv_sems.at[s, c],
                device_id=(partners[s],),
                device_id_type=pl.DeviceIdType.MESH,
            )
            rdma.start()
            return rdma

        def finish(s, c, acc, rdma):
            rdma.wait()
            return acc + recv_ref[s, c][...].astype(jnp.float32)

        acc_t = compute_chunk(0, 0)
        x_t = exchange(0, 0, acc_t)
        acc_b = compute_chunk(1, HEADS_PER)
        x_b = exchange(0, 1, acc_b)
        acc_t = finish(0, 0, acc_t, x_t)
        y_t = exchange(1, 0, acc_t)
        acc_b = finish(0, 1, acc_b, x_b)
        y_b = exchange(1, 1, acc_b)
        acc_t = finish(1, 0, acc_t, y_t)
        z_t = exchange(2, 0, acc_t)
        acc_b = finish(1, 1, acc_b, y_b)
        z_b = exchange(2, 1, acc_b)
        acc_t = finish(2, 0, acc_t, z_t)
        out_ref[0, 0:HALF, :] = acc_t
        acc_b = finish(2, 1, acc_b, z_b)
        out_ref[0, HALF:SQ, :] = acc_b

    out = pl.pallas_call(
        body,
        out_shape=jax.ShapeDtypeStruct((1, SQ, DM), jnp.float32),
        in_specs=[
            pl.BlockSpec(memory_space=pltpu.VMEM),
            pl.BlockSpec(memory_space=pltpu.VMEM),
            pl.BlockSpec(memory_space=pl.ANY),
            pl.BlockSpec(memory_space=pl.ANY),
            pl.BlockSpec(memory_space=pltpu.VMEM),
        ],
        out_specs=pl.BlockSpec(memory_space=pltpu.VMEM),
        scratch_shapes=[
            pltpu.VMEM((2, 2, SKV, DH), jnp.float32),
            pltpu.SemaphoreType.DMA((2, 2)),
            pltpu.VMEM((3, 2, HALF, DM), jnp.bfloat16),
            pltpu.VMEM((3, 2, HALF, DM), jnp.bfloat16),
            pltpu.SemaphoreType.DMA((3, 2)),
            pltpu.SemaphoreType.DMA((3, 2)),
        ],
        compiler_params=pltpu.CompilerParams(collective_id=0),
    )(x, Wq, K_ext, V_ext, Wo)
    return out


# device time: 48203 ns/iter; 1.2981x vs baseline; 1.2981x over previous
import jax
import jax.numpy as jnp
from jax import lax
from jax.experimental import pallas as pl
from jax.experimental.pallas import tpu as pltpu

N_DEV = 8
HEADS_PER = 8
SQ = 256
HALF = SQ // 2
SKV = 4096
DH = 128
DM = HEADS_PER * DH
BLK = 64
SCALE = 0.08838834764831843


def kernel(x, Wq, K_ext, V_ext, Wo):
    XOR_STAGES = (1, 3, 4)

    def body(x_ref, wq_ref, k_any, v_any, wo_ref, out_ref,
             kv_bufs, kv_sems, send_ref, recv_ref, send_sems, recv_sems):
        my_pos = lax.axis_index("i")
        partners = [jnp.bitwise_xor(my_pos, c) for c in XOR_STAGES]

        barrier_sem = pltpu.get_barrier_semaphore()
        for nbr in partners:
            pl.semaphore_signal(
                barrier_sem, inc=1,
                device_id=(nbr,), device_id_type=pl.DeviceIdType.MESH,
            )
        pl.semaphore_wait(barrier_sem, len(partners))

        def start_kv(h, slot):
            head = my_pos * HEADS_PER + h
            kcp = pltpu.make_async_copy(
                k_any.at[0, :, head, :], kv_bufs.at[slot, 0],
                kv_sems.at[slot, 0])
            vcp = pltpu.make_async_copy(
                v_any.at[0, :, head, :], kv_bufs.at[slot, 1],
                kv_sems.at[slot, 1])
            kcp.start()
            vcp.start()
            return kcp, vcp

        pending = start_kv(0, 0)

        xb = x_ref[0].astype(jnp.bfloat16)
        wqb = wq_ref[...].astype(jnp.bfloat16)
        wob = wo_ref[...].astype(jnp.bfloat16)
        q = jnp.dot(xb, wqb, preferred_element_type=jnp.float32)
        qs = (q * SCALE).astype(jnp.bfloat16)

        rowb = lax.broadcasted_iota(jnp.int32, (SQ, SKV), 0) // BLK
        colb = lax.broadcasted_iota(jnp.int32, (SQ, SKV), 1) // BLK
        keep = (rowb == colb) | (colb == 0) | (lax.rem(rowb + colb, 3) == 0)
        bias = jnp.where(keep, 0.0, -1e9).astype(jnp.float32)

        acc = jnp.zeros((SQ, DM), jnp.float32)
        for h in range(HEADS_PER):
            slot = h % 2
            pending[0].wait()
            pending[1].wait()
            if h + 1 < HEADS_PER:
                pending = start_kv(h + 1, (h + 1) % 2)
            kh = kv_bufs[slot, 0].astype(jnp.bfloat16)
            vh = kv_bufs[slot, 1].astype(jnp.bfloat16)
            qh = qs[:, h * DH:(h + 1) * DH]
            s = lax.dot_general(
                qh, kh, (((1,), (1,)), ((), ())),
                preferred_element_type=jnp.float32)
            w = jnp.exp(s + bias)
            denom = jnp.sum(w, axis=-1, keepdims=True)
            ctx = jnp.dot(w.astype(jnp.bfloat16), vh,
                          preferred_element_type=jnp.float32)
            ctx = (ctx / denom).astype(jnp.bfloat16)
            acc = acc + jnp.dot(ctx, wob[h * DH:(h + 1) * DH, :],
                                preferred_element_type=jnp.float32)

        def exchange(s, c, acc):
            send_ref[s, c] = acc.astype(jnp.bfloat16)
            rdma = pltpu.make_async_remote_copy(
                src_ref=send_ref.at[s, c],
                dst_ref=recv_ref.at[s, c],
                send_sem=send_sems.at[s, c],
                recv_sem=recv_sems.at[s, c],
                device_id=(partners[s],),
                device_id_type=pl.DeviceIdType.MESH,
            )
            rdma.start()
            return rdma

        def finish(s, c, acc, rdma):
            rdma.wait()
            return acc + recv_ref[s, c][...].astype(jnp.float32)

        acc_t = acc[0:HALF]
        acc_b = acc[HALF:SQ]
        x_t = exchange(0, 0, acc_t)
        x_b = exchange(0, 1, acc_b)
        acc_t = finish(0, 0, acc_t, x_t)
        y_t = exchange(1, 0, acc_t)
        acc_b = finish(0, 1, acc_b, x_b)
        y_b = exchange(1, 1, acc_b)
        acc_t = finish(1, 0, acc_t, y_t)
        z_t = exchange(2, 0, acc_t)
        acc_b = finish(1, 1, acc_b, y_b)
        z_b = exchange(2, 1, acc_b)
        acc_t = finish(2, 0, acc_t, z_t)
        out_ref[0, 0:HALF, :] = acc_t
        acc_b = finish(2, 1, acc_b, z_b)
        out_ref[0, HALF:SQ, :] = acc_b

    out = pl.pallas_call(
        body,
        out_shape=jax.ShapeDtypeStruct((1, SQ, DM), jnp.float32),
        in_specs=[
            pl.BlockSpec(memory_space=pltpu.VMEM),
            pl.BlockSpec(memory_space=pltpu.VMEM),
            pl.BlockSpec(memory_space=pl.ANY),
            pl.BlockSpec(memory_space=pl.ANY),
            pl.BlockSpec(memory_space=pltpu.VMEM),
        ],
        out_specs=pl.BlockSpec(memory_space=pltpu.VMEM),
        scratch_shapes=[
            pltpu.VMEM((2, 2, SKV, DH), jnp.float32),
            pltpu.SemaphoreType.DMA((2, 2)),
            pltpu.VMEM((3, 2, HALF, DM), jnp.bfloat16),
            pltpu.VMEM((3, 2, HALF, DM), jnp.bfloat16),
            pltpu.SemaphoreType.DMA((3, 2)),
            pltpu.SemaphoreType.DMA((3, 2)),
        ],
        compiler_params=pltpu.CompilerParams(
            collective_id=0, vmem_limit_bytes=62 * 1024 * 1024),
    )(x, Wq, K_ext, V_ext, Wo)
    return out
